# Initial kernel scaffold; baseline (speedup 1.0000x reference)
import jax
import jax.numpy as jnp
from jax.experimental import pallas as pl


def kernel(input_ids, table):
    raise NotImplementedError("write your pallas kernel here")


# speedup vs baseline: 5.1878x; 5.1878x over previous
import functools

import jax
import jax.numpy as jnp
from jax import lax
from jax.experimental import pallas as pl
from jax.experimental.pallas import tpu as pltpu
from jax.experimental.pallas import tpu_sc as plsc

VOCAB = 100000
DIM = 128
B = 4096
L = 200

_info = plsc.get_sparse_core_info()
NC, NS = _info.num_cores, _info.num_subcores
NW = NC * NS

TOTAL = B * L
PER_W = TOTAL // NW
CHUNK = 128
N_CHUNKS = PER_W // CHUNK


def _make_gather():
    mesh = plsc.VectorSubcoreMesh(core_axis_name="c", subcore_axis_name="s")

    @functools.partial(
        pl.kernel,
        mesh=mesh,
        out_type=jax.ShapeDtypeStruct((TOTAL, DIM), jnp.float32),
        scratch_types=[
            pltpu.VMEM((CHUNK,), jnp.int32),
            pltpu.VMEM((CHUNK, DIM), jnp.float32),
            pltpu.SemaphoreType.DMA,
        ],
    )
    def gather_kernel(table_hbm, ids_hbm, out_hbm, idx_v, rows_v, sem):
        wid = lax.axis_index("s") * NC + lax.axis_index("c")
        base = wid * PER_W

        def body(g, _):
            off = base + g * CHUNK
            pltpu.sync_copy(ids_hbm.at[pl.ds(off, CHUNK)], idx_v)
            pltpu.async_copy(table_hbm.at[idx_v], rows_v, sem).wait()
            pltpu.sync_copy(rows_v, out_hbm.at[pl.ds(off, CHUNK)])
            return 0

        lax.fori_loop(0, N_CHUNKS, body, 0)

    return gather_kernel


_gather = _make_gather()


def kernel(input_ids, table):
    ids_flat = input_ids.reshape(TOTAL).astype(jnp.int32)
    out_flat = _gather(table, ids_flat)
    return out_flat.reshape(B, L, DIM)


# speedup vs baseline: 8.4962x; 1.6377x over previous
import functools

import jax
import jax.numpy as jnp
from jax import lax
from jax.experimental import pallas as pl
from jax.experimental.pallas import tpu as pltpu
from jax.experimental.pallas import tpu_sc as plsc

VOCAB = 100000
DIM = 128
B = 4096
L = 200

_info = plsc.get_sparse_core_info()
NC, NS = _info.num_cores, _info.num_subcores
NW = NC * NS

TOTAL = B * L
PER_W = TOTAL // NW
CHUNK = 128
N_CHUNKS = PER_W // CHUNK
HALF = N_CHUNKS // 2


def _make_gather():
    mesh = plsc.VectorSubcoreMesh(core_axis_name="c", subcore_axis_name="s")

    @functools.partial(
        pl.kernel,
        mesh=mesh,
        out_type=jax.ShapeDtypeStruct((TOTAL, DIM), jnp.float32),
        scratch_types=[
            pltpu.VMEM((N_CHUNKS, CHUNK), jnp.int32),
            pltpu.VMEM((CHUNK, DIM), jnp.float32),
            pltpu.VMEM((CHUNK, DIM), jnp.float32),
            pltpu.SemaphoreType.DMA,
            pltpu.SemaphoreType.DMA,
            pltpu.SemaphoreType.DMA,
            pltpu.SemaphoreType.DMA,
        ],
    )
    def gather_kernel(table_hbm, ids_hbm, out_hbm, idx_all, rows0, rows1,
                      sem_g0, sem_g1, sem_s0, sem_s1):
        wid = lax.axis_index("s") * NC + lax.axis_index("c")
        base = wid * PER_W

        pltpu.sync_copy(ids_hbm.at[wid], idx_all)

        def out_at(k):
            return out_hbm.at[pl.ds(base + k * CHUNK, CHUNK)]

        pltpu.async_copy(table_hbm.at[idx_all.at[0]], rows0, sem_g0)
        pltpu.async_copy(table_hbm.at[idx_all.at[1]], rows1, sem_g1)
        pltpu.make_async_copy(table_hbm.at[idx_all.at[0]], rows0, sem_g0).wait()
        pltpu.async_copy(rows0, out_at(0), sem_s0)
        pltpu.make_async_copy(table_hbm.at[idx_all.at[1]], rows1, sem_g1).wait()
        pltpu.async_copy(rows1, out_at(1), sem_s1)

        def body(j, _):
            k0 = 2 * j
            k1 = k0 + 1
            pltpu.make_async_copy(rows0, out_at(k0 - 2), sem_s0).wait()
            pltpu.async_copy(table_hbm.at[idx_all.at[k0]], rows0, sem_g0)
            pltpu.make_async_copy(rows1, out_at(k1 - 2), sem_s1).wait()
            pltpu.async_copy(table_hbm.at[idx_all.at[k1]], rows1, sem_g1)
            pltpu.make_async_copy(table_hbm.at[idx_all.at[k0]], rows0, sem_g0).wait()
            pltpu.async_copy(rows0, out_at(k0), sem_s0)
            pltpu.make_async_copy(table_hbm.at[idx_all.at[k1]], rows1, sem_g1).wait()
            pltpu.async_copy(rows1, out_at(k1), sem_s1)
            return 0

        lax.fori_loop(1, HALF, body, 0)

        pltpu.make_async_copy(rows0, out_at(N_CHUNKS - 2), sem_s0).wait()
        pltpu.make_async_copy(rows1, out_at(N_CHUNKS - 1), sem_s1).wait()

    return gather_kernel


_gather = _make_gather()


def kernel(input_ids, table):
    ids = input_ids.reshape(NW, N_CHUNKS, CHUNK).astype(jnp.int32)
    out_flat = _gather(table, ids)
    return out_flat.reshape(B, L, DIM)


# speedup vs baseline: 9.1629x; 1.0785x over previous
import functools

import jax
import jax.numpy as jnp
from jax import lax
from jax.experimental import pallas as pl
from jax.experimental.pallas import tpu as pltpu
from jax.experimental.pallas import tpu_sc as plsc

VOCAB = 100000
DIM = 128
B = 4096
L = 200

_info = plsc.get_sparse_core_info()
NC, NS = _info.num_cores, _info.num_subcores
NW = NC * NS

TOTAL = B * L
PER_W = TOTAL // NW
CHUNK = 128
N_CHUNKS = PER_W // CHUNK
NBUF = 4
N_GROUPS = N_CHUNKS // NBUF


def _make_gather():
    mesh = plsc.VectorSubcoreMesh(core_axis_name="c", subcore_axis_name="s")

    @functools.partial(
        pl.kernel,
        mesh=mesh,
        out_type=jax.ShapeDtypeStruct((TOTAL, DIM), jnp.float32),
        scratch_types=[
            pltpu.VMEM((N_CHUNKS, CHUNK), jnp.int32),
            pltpu.VMEM((NBUF, CHUNK, DIM), jnp.float32),
            pltpu.SemaphoreType.DMA((NBUF,)),
            pltpu.SemaphoreType.DMA((NBUF,)),
        ],
    )
    def gather_kernel(table_hbm, ids_hbm, out_hbm, idx_all, rows, sem_g, sem_s):
        wid = lax.axis_index("s") * NC + lax.axis_index("c")
        base = wid * PER_W

        pltpu.sync_copy(ids_hbm.at[wid], idx_all)

        def gather(k, b):
            pltpu.async_copy(table_hbm.at[idx_all.at[k]], rows.at[b], sem_g.at[b])

        def wait_gather(k, b):
            pltpu.make_async_copy(
                table_hbm.at[idx_all.at[k]], rows.at[b], sem_g.at[b]).wait()

        def store(k, b):
            pltpu.async_copy(
                rows.at[b], out_hbm.at[pl.ds(base + k * CHUNK, CHUNK)], sem_s.at[b])

        def wait_store(k, b):
            pltpu.make_async_copy(
                rows.at[b], out_hbm.at[pl.ds(base + k * CHUNK, CHUNK)],
                sem_s.at[b]).wait()

        for b in range(NBUF):
            gather(b, b)
        for b in range(NBUF):
            wait_gather(b, b)
            store(b, b)

        def body(j, _):
            k0 = j * NBUF
            for b in range(NBUF):
                k = k0 + b
                wait_store(k - NBUF, b)
                gather(k, b)
            for b in range(NBUF):
                k = k0 + b
                wait_gather(k, b)
                store(k, b)
            return 0

        lax.fori_loop(1, N_GROUPS, body, 0)

        for b in range(NBUF):
            wait_store(N_CHUNKS - NBUF + b, b)

    return gather_kernel


_gather = _make_gather()


def kernel(input_ids, table):
    ids = input_ids.reshape(NW, N_CHUNKS, CHUNK).astype(jnp.int32)
    out_flat = _gather(table, ids)
    return out_flat.reshape(B, L, DIM)
